# Initial kernel scaffold; baseline (speedup 1.0000x reference)
import jax
import jax.numpy as jnp
from jax.experimental import pallas as pl


def kernel(xyz, features, curvature, W1, b1, W2, b2, W3, b3):
    raise NotImplementedError("write your pallas kernel here")


# speedup vs baseline: 11.0605x; 11.0605x over previous
import functools

import numpy as np

import jax
import jax.numpy as jnp
from jax import lax
from jax.experimental import pallas as pl
from jax.experimental.pallas import tpu as pltpu
from jax.experimental.pallas import tpu_sc as plsc

_B, _N, _C = 2, 8192, 32
_S, _RADIUS, _NS = 1024, 0.2, 32
_R2 = float(np.float32(0.2 ** 2))
_ROWS, _LANES = _N // 128, 128


def _fps_body(xyz_ref, curv_ref, idx_ref, nxyz_ref, ncurv_ref, pw_ref):
    px = xyz_ref[0, 0, :, :]
    py = xyz_ref[0, 1, :, :]
    pz = xyz_ref[0, 2, :, :]
    nrm = jnp.sqrt(px * px + py * py + pz * pz)
    pw_ref[...] = jnp.where(nrm > 0.7, 0.0, curv_ref[0])
    pw = pw_ref[...]

    rows = lax.broadcasted_iota(jnp.int32, (_ROWS, _LANES), 0)
    lanes = lax.broadcasted_iota(jnp.int32, (_ROWS, _LANES), 1)
    gidx = rows * _LANES + lanes
    lane1 = lax.broadcasted_iota(jnp.int32, (1, _LANES), 1)

    def pick(plane_ref, pre, r, l):
        rowv = plane_ref[pre + (pl.ds(r, 1), slice(None))]
        return jnp.sum(jnp.where(lane1 == l, rowv, 0.0))

    def body(i, carry):
        dist, g = carry
        r = g // _LANES
        l = g % _LANES
        cx = pick(xyz_ref, (0, 0), r, l)
        cy = pick(xyz_ref, (0, 1), r, l)
        cz = pick(xyz_ref, (0, 2), r, l)
        cw = pick(pw_ref, (), r, l)
        idx_ref[0, 0, i] = g
        nxyz_ref[0, 0, i] = cx
        nxyz_ref[0, 1, i] = cy
        nxyz_ref[0, 2, i] = cz
        ncurv_ref[0, 0, i] = cw
        dx = px - cx
        dy = py - cy
        dz = pz - cz
        dw = pw - cw
        d = ((dx * dx + dy * dy) + dz * dz) + dw * dw
        dist = jnp.minimum(dist, d)
        m = jnp.max(dist)
        g2 = jnp.min(jnp.where(dist == m, gidx, _N))
        return dist, g2

    dist0 = jnp.full((_ROWS, _LANES), 1e10, dtype=jnp.float32)
    lax.fori_loop(0, _S, body, (dist0, jnp.int32(0)))


def _fps(xyz, curv):
    xyz_pl = xyz.transpose(0, 2, 1).reshape(_B, 3, _ROWS, _LANES)
    curv_pl = curv.reshape(_B, _ROWS, _LANES)
    out = pl.pallas_call(
        _fps_body,
        grid=(_B,),
        in_specs=[
            pl.BlockSpec((1, 3, _ROWS, _LANES), lambda b: (b, 0, 0, 0)),
            pl.BlockSpec((1, _ROWS, _LANES), lambda b: (b, 0, 0)),
        ],
        out_specs=[
            pl.BlockSpec((1, 1, _S), lambda b: (b, 0, 0), memory_space=pltpu.SMEM),
            pl.BlockSpec((1, 3, _S), lambda b: (b, 0, 0), memory_space=pltpu.SMEM),
            pl.BlockSpec((1, 1, _S), lambda b: (b, 0, 0), memory_space=pltpu.SMEM),
        ],
        out_shape=[
            jax.ShapeDtypeStruct((_B, 1, _S), jnp.int32),
            jax.ShapeDtypeStruct((_B, 3, _S), jnp.float32),
            jax.ShapeDtypeStruct((_B, 1, _S), jnp.float32),
        ],
        scratch_shapes=[pltpu.VMEM((_ROWS, _LANES), jnp.float32)],
    )(xyz_pl, curv_pl)
    return out[0].reshape(_B, _S), out[1], out[2].reshape(_B, _S)


_QPT = _S // 16
_SEG = 1024
_NSEG = _N // _SEG
_TSB = 128


def _d2_body(nx_ref, xt_ref, out_ref):
    a = nx_ref[0]
    bt = xt_ref[0]
    qx = lax.dot_general(a, bt, (((1,), (0,)), ((), ())),
                         preferred_element_type=jnp.float32)
    q2 = (a[:, 0:1] * a[:, 0:1] + a[:, 1:2] * a[:, 1:2]) + a[:, 2:3] * a[:, 2:3]
    x2 = (bt[0:1, :] * bt[0:1, :] + bt[1:2, :] * bt[1:2, :]) + bt[2:3, :] * bt[2:3, :]
    out_ref[0] = (q2 + x2) - 2.0 * qx


def _d2(new_xyz, xyz_t):
    return pl.pallas_call(
        _d2_body,
        grid=(_B, _S // _TSB),
        in_specs=[
            pl.BlockSpec((1, _TSB, 3), lambda b, t: (b, t, 0)),
            pl.BlockSpec((1, 3, _N), lambda b, t: (b, 0, 0)),
        ],
        out_specs=pl.BlockSpec((1, _TSB, _N), lambda b, t: (b, t, 0)),
        out_shape=jax.ShapeDtypeStruct((_B, _S, _N), jnp.float32),
    )(new_xyz, xyz_t)


def _ballq_body(d2_ref, out_ref, dbuf, ob):
    c = lax.axis_index("c")
    s = lax.axis_index("s")
    iota = lax.iota(jnp.int32, 16)
    big = jnp.full((16,), _N, dtype=jnp.int32)

    def q_body(j, _):
        jv = jnp.full((16,), j, dtype=jnp.int32)
        qid = s * _QPT + j
        ob[j, pl.ds(0, 16)] = big
        ob[j, pl.ds(16, 16)] = big

        def seg_cond(carry):
            seg, cnt = carry
            return (seg < _NSEG) & (cnt < _NS)

        def seg_body(carry):
            seg, cnt = carry
            pltpu.sync_copy(
                d2_ref.at[c, pl.ds(qid, 1), pl.ds(seg * _SEG, _SEG)], dbuf)

            def cond(carry2):
                ch, cnt2 = carry2
                return (ch < _SEG // 16) & (cnt2 < _NS)

            def scan(carry2):
                ch, cnt2 = carry2
                d2c = dbuf[0, pl.ds(ch * 16, 16)]
                m = d2c < _R2
                rank = plsc.cumsum(jnp.where(m, 1, 0))
                pos = cnt2 + rank - 1
                sel = m & (pos < _NS)
                lanev = (seg * _SEG + ch * 16) + iota
                plsc.store_scatter(ob, [jv, pos], lanev, mask=sel)
                nv = lax.reduce_max(rank, axes=(0,))
                return ch + 1, cnt2 + nv

            _, cnt = lax.while_loop(cond, scan, (jnp.int32(0), cnt))
            return seg + 1, cnt

        _, cnt = lax.while_loop(seg_cond, seg_body, (jnp.int32(0), jnp.int32(0)))
        o0 = ob[j, pl.ds(0, 16)]
        o1 = ob[j, pl.ds(16, 16)]
        padv = lax.reduce_min(o0, axes=(0,))
        ob[j, pl.ds(0, 16)] = jnp.where(iota < cnt, o0, padv)
        ob[j, pl.ds(16, 16)] = jnp.where(iota + 16 < cnt, o1, padv)
        return 0

    lax.fori_loop(0, _QPT, q_body, 0)
    pltpu.sync_copy(ob, out_ref.at[c, pl.ds(s * _QPT, _QPT)])


def _ball_query(xyz, new_xyz):
    d2 = _d2(new_xyz, xyz.transpose(0, 2, 1))
    mesh = plsc.VectorSubcoreMesh(core_axis_name="c", subcore_axis_name="s",
                                  num_cores=2, num_subcores=16)
    f = pl.kernel(
        _ballq_body,
        out_type=jax.ShapeDtypeStruct((_B, _S, _NS), jnp.int32),
        mesh=mesh,
        scratch_types=[
            pltpu.VMEM((1, _SEG), jnp.float32),
            pltpu.VMEM((_QPT, _NS), jnp.int32),
        ],
        compiler_params=pltpu.CompilerParams(needs_layout_passes=False),
    )
    return f(d2)


_D = 128
_RPT = _S * _NS // 16
_GC = 128
_NCH = _RPT // _GC


def _gather_body(tab_ref, gidx_ref, out_ref, idxv, buf0, sem0):
    c = lax.axis_index("c")
    s = lax.axis_index("s")
    pltpu.sync_copy(gidx_ref.at[c, pl.ds(s * _NCH, _NCH)], idxv)
    base = s * _RPT
    src = tab_ref.at[c]

    def loop(i, _):
        pltpu.async_copy(src.at[idxv.at[i]], buf0, sem0).wait()
        pltpu.sync_copy(buf0, out_ref.at[c, pl.ds(base + i * _GC, _GC)])
        return 0

    lax.fori_loop(0, _NCH, loop, 0)


def _gather(table, gidx):
    gidx_r = gidx.reshape(_B, _S * _NS // 128, 128)
    mesh = plsc.VectorSubcoreMesh(core_axis_name="c", subcore_axis_name="s",
                                  num_cores=2, num_subcores=16)
    f = pl.kernel(
        _gather_body,
        out_type=jax.ShapeDtypeStruct((_B, _S * _NS, _D), jnp.float32),
        mesh=mesh,
        scratch_types=[
            pltpu.VMEM((_NCH, _GC), jnp.int32),
            pltpu.VMEM((_GC, _D), jnp.float32),
            pltpu.SemaphoreType.DMA,
        ],
        compiler_params=pltpu.CompilerParams(needs_layout_passes=False),
    )
    return f(table, gidx_r)


_TS = 128


def _mlp_body(g_ref, nx_ref, w1_ref, b1_ref, w2_ref, b2_ref, w3_ref, b3_ref,
              out_ref):
    g = g_ref[0]
    nx = nx_ref[0]
    w1 = w1_ref[...]
    corr = jnp.dot(nx, w1_ref[0:3, :], preferred_element_type=jnp.float32)
    corr = jnp.reshape(
        jnp.broadcast_to(corr[:, None, :], (_TS, _NS, 64)), (_TS * _NS, 64))
    h = jnp.dot(g, w1, preferred_element_type=jnp.float32)
    h = jnp.maximum(h + b1_ref[...] - corr, 0.0)
    h = jnp.dot(h, w2_ref[...], preferred_element_type=jnp.float32)
    h = jnp.maximum(h + b2_ref[...], 0.0)
    h = jnp.dot(h, w3_ref[...], preferred_element_type=jnp.float32)
    h = jnp.maximum(h + b3_ref[...], 0.0)
    out_ref[0] = jnp.max(jnp.reshape(h, (_TS, _NS, 128)), axis=1)


def _mlp(g, new_xyz, W1p, b1, W2, b2, W3, b3):
    return pl.pallas_call(
        _mlp_body,
        grid=(_B, _S // _TS),
        in_specs=[
            pl.BlockSpec((1, _TS * _NS, _D), lambda b, t: (b, t, 0)),
            pl.BlockSpec((1, _TS, 3), lambda b, t: (b, t, 0)),
            pl.BlockSpec((_D, 64), lambda b, t: (0, 0)),
            pl.BlockSpec((1, 64), lambda b, t: (0, 0)),
            pl.BlockSpec((64, 64), lambda b, t: (0, 0)),
            pl.BlockSpec((1, 64), lambda b, t: (0, 0)),
            pl.BlockSpec((64, 128), lambda b, t: (0, 0)),
            pl.BlockSpec((1, 128), lambda b, t: (0, 0)),
        ],
        out_specs=pl.BlockSpec((1, _TS, 128), lambda b, t: (b, t, 0)),
        out_shape=jax.ShapeDtypeStruct((_B, _S, 128), jnp.float32),
    )(g, new_xyz, W1p, b1.reshape(1, 64), W2, b2.reshape(1, 64),
      W3, b3.reshape(1, 128))


def kernel(xyz, features, curvature, W1, b1, W2, b2, W3, b3):
    idx, nxyz_t, new_curv = _fps(xyz, curvature)
    new_xyz = nxyz_t.transpose(0, 2, 1)
    gidx = _ball_query(xyz, new_xyz)
    table = jnp.concatenate(
        [xyz, features.transpose(0, 2, 1),
         jnp.zeros((_B, _N, _D - 3 - _C), jnp.float32)], axis=-1)
    g = _gather(table, gidx)
    W1p = jnp.zeros((_D, 64), jnp.float32).at[0:3 + _C].set(W1)
    nf = _mlp(g, new_xyz, W1p, b1, W2, b2, W3, b3)
    return new_xyz, nf.transpose(0, 2, 1), new_curv


# speedup vs baseline: 12.3589x; 1.1174x over previous
import functools

import numpy as np

import jax
import jax.numpy as jnp
from jax import lax
from jax.experimental import pallas as pl
from jax.experimental.pallas import tpu as pltpu
from jax.experimental.pallas import tpu_sc as plsc

_B, _N, _C = 2, 8192, 32
_S, _RADIUS, _NS = 1024, 0.2, 32
_R2 = float(np.float32(0.2 ** 2))
_ROWS, _LANES = _N // 128, 128


def _fps_body(xyz_ref, curv_ref, idx_ref, nxyz_ref, ncurv_ref, pw_ref):
    planes = []
    for b in range(_B):
        px = xyz_ref[b, 0, :, :]
        py = xyz_ref[b, 1, :, :]
        pz = xyz_ref[b, 2, :, :]
        nrm = jnp.sqrt(px * px + py * py + pz * pz)
        pw_ref[b] = jnp.where(nrm > 0.7, 0.0, curv_ref[b])
        planes.append((px, py, pz, pw_ref[b]))

    rows = lax.broadcasted_iota(jnp.int32, (_ROWS, _LANES), 0)
    lanes = lax.broadcasted_iota(jnp.int32, (_ROWS, _LANES), 1)
    gidx = rows * _LANES + lanes
    lane1 = lax.broadcasted_iota(jnp.int32, (1, _LANES), 1)

    def pick(plane_ref, pre, r, l):
        rowv = plane_ref[pre + (pl.ds(r, 1), slice(None))]
        return jnp.sum(jnp.where(lane1 == l, rowv, 0.0))

    def body(i, carry):
        dists, gs = carry
        new_dists, new_gs = [], []
        for b in range(_B):
            dist, g = dists[b], gs[b]
            px, py, pz, pw = planes[b]
            r = g // _LANES
            l = g % _LANES
            cx = pick(xyz_ref, (b, 0), r, l)
            cy = pick(xyz_ref, (b, 1), r, l)
            cz = pick(xyz_ref, (b, 2), r, l)
            cw = pick(pw_ref, (b,), r, l)
            idx_ref[b, 0, i] = g
            nxyz_ref[b, 0, i] = cx
            nxyz_ref[b, 1, i] = cy
            nxyz_ref[b, 2, i] = cz
            ncurv_ref[b, 0, i] = cw
            dx = px - cx
            dy = py - cy
            dz = pz - cz
            dw = pw - cw
            d = ((dx * dx + dy * dy) + dz * dz) + dw * dw
            dist = jnp.minimum(dist, d)
            m = jnp.max(dist)
            g2 = jnp.min(jnp.where(dist == m, gidx, _N))
            new_dists.append(dist)
            new_gs.append(g2)
        return tuple(new_dists), tuple(new_gs)

    dist0 = jnp.full((_ROWS, _LANES), 1e10, dtype=jnp.float32)
    lax.fori_loop(0, _S, body,
                  ((dist0,) * _B, (jnp.int32(0),) * _B))


def _fps(xyz, curv):
    xyz_pl = xyz.transpose(0, 2, 1).reshape(_B, 3, _ROWS, _LANES)
    curv_pl = curv.reshape(_B, _ROWS, _LANES)
    out = pl.pallas_call(
        _fps_body,
        in_specs=[
            pl.BlockSpec((_B, 3, _ROWS, _LANES), lambda: (0, 0, 0, 0)),
            pl.BlockSpec((_B, _ROWS, _LANES), lambda: (0, 0, 0)),
        ],
        out_specs=[
            pl.BlockSpec((_B, 1, _S), lambda: (0, 0, 0), memory_space=pltpu.SMEM),
            pl.BlockSpec((_B, 3, _S), lambda: (0, 0, 0), memory_space=pltpu.SMEM),
            pl.BlockSpec((_B, 1, _S), lambda: (0, 0, 0), memory_space=pltpu.SMEM),
        ],
        out_shape=[
            jax.ShapeDtypeStruct((_B, 1, _S), jnp.int32),
            jax.ShapeDtypeStruct((_B, 3, _S), jnp.float32),
            jax.ShapeDtypeStruct((_B, 1, _S), jnp.float32),
        ],
        scratch_shapes=[pltpu.VMEM((_B, _ROWS, _LANES), jnp.float32)],
    )(xyz_pl, curv_pl)
    return out[0].reshape(_B, _S), out[1], out[2].reshape(_B, _S)


_QPT = _S // 16
_SEG = 1024
_NSEG = _N // _SEG
_TSB = 128


def _d2_body(nx_ref, xt_ref, out_ref):
    a = nx_ref[0]
    bt = xt_ref[0]
    qx = lax.dot_general(a, bt, (((1,), (0,)), ((), ())),
                         preferred_element_type=jnp.float32)
    q2 = (a[:, 0:1] * a[:, 0:1] + a[:, 1:2] * a[:, 1:2]) + a[:, 2:3] * a[:, 2:3]
    x2 = (bt[0:1, :] * bt[0:1, :] + bt[1:2, :] * bt[1:2, :]) + bt[2:3, :] * bt[2:3, :]
    out_ref[0] = (q2 + x2) - 2.0 * qx


def _d2(new_xyz, xyz_t):
    return pl.pallas_call(
        _d2_body,
        grid=(_B, _S // _TSB),
        in_specs=[
            pl.BlockSpec((1, _TSB, 3), lambda b, t: (b, t, 0)),
            pl.BlockSpec((1, 3, _N), lambda b, t: (b, 0, 0)),
        ],
        out_specs=pl.BlockSpec((1, _TSB, _N), lambda b, t: (b, t, 0)),
        out_shape=jax.ShapeDtypeStruct((_B, _S, _N), jnp.float32),
    )(new_xyz, xyz_t)


def _ballq_body(d2_ref, out_ref, dball, dbuf, ob):
    c = lax.axis_index("c")
    s = lax.axis_index("s")
    iota = lax.iota(jnp.int32, 16)
    big = jnp.full((16,), _N, dtype=jnp.int32)
    pltpu.sync_copy(d2_ref.at[c, pl.ds(s * _QPT, _QPT), pl.ds(0, _SEG)], dball)

    def q_body(j, _):
        jv = jnp.full((16,), j, dtype=jnp.int32)
        qid = s * _QPT + j
        ob[j, pl.ds(0, 16)] = big
        ob[j, pl.ds(16, 16)] = big

        def scan_row(row_ref, row_pre, seg_base, cnt0):
            def cond(carry2):
                ch, cnt2 = carry2
                return (ch < _SEG // 16) & (cnt2 < _NS)

            def scan(carry2):
                ch, cnt2 = carry2
                d2c = row_ref[row_pre + (pl.ds(ch * 16, 16),)]
                m = d2c < _R2
                rank = plsc.cumsum(jnp.where(m, 1, 0))
                pos = cnt2 + rank - 1
                sel = m & (pos < _NS)
                lanev = (seg_base + ch * 16) + iota
                plsc.store_scatter(ob, [jv, pos], lanev, mask=sel)
                nv = lax.reduce_max(rank, axes=(0,))
                return ch + 1, cnt2 + nv

            _, cnt = lax.while_loop(cond, scan, (jnp.int32(0), cnt0))
            return cnt

        cnt = scan_row(dball, (j,), jnp.int32(0), jnp.int32(0))

        def seg_cond(carry):
            seg, cnt2 = carry
            return (seg < _NSEG) & (cnt2 < _NS)

        def seg_body(carry):
            seg, cnt2 = carry
            pltpu.sync_copy(
                d2_ref.at[c, pl.ds(qid, 1), pl.ds(seg * _SEG, _SEG)], dbuf)
            cnt2 = scan_row(dbuf, (0,), seg * _SEG, cnt2)
            return seg + 1, cnt2

        _, cnt = lax.while_loop(seg_cond, seg_body, (jnp.int32(1), cnt))
        o0 = ob[j, pl.ds(0, 16)]
        o1 = ob[j, pl.ds(16, 16)]
        padv = lax.reduce_min(o0, axes=(0,))
        ob[j, pl.ds(0, 16)] = jnp.where(iota < cnt, o0, padv)
        ob[j, pl.ds(16, 16)] = jnp.where(iota + 16 < cnt, o1, padv)
        return 0

    lax.fori_loop(0, _QPT, q_body, 0)
    pltpu.sync_copy(ob, out_ref.at[c, pl.ds(s * _QPT, _QPT)])


def _ball_query(xyz, new_xyz):
    d2 = _d2(new_xyz, xyz.transpose(0, 2, 1))
    mesh = plsc.VectorSubcoreMesh(core_axis_name="c", subcore_axis_name="s",
                                  num_cores=2, num_subcores=16)
    f = pl.kernel(
        _ballq_body,
        out_type=jax.ShapeDtypeStruct((_B, _S, _NS), jnp.int32),
        mesh=mesh,
        scratch_types=[
            pltpu.VMEM((_QPT, _SEG), jnp.float32),
            pltpu.VMEM((1, _SEG), jnp.float32),
            pltpu.VMEM((_QPT, _NS), jnp.int32),
        ],
        compiler_params=pltpu.CompilerParams(needs_layout_passes=False),
    )
    return f(d2)


_D = 128
_RPT = _S * _NS // 16
_GC = 128
_NCH = _RPT // _GC


def _gather_body(tab_ref, gidx_ref, out_ref, idxv, buf0, sem0):
    c = lax.axis_index("c")
    s = lax.axis_index("s")
    pltpu.sync_copy(gidx_ref.at[c, pl.ds(s * _NCH, _NCH)], idxv)
    base = s * _RPT
    src = tab_ref.at[c]

    def loop(i, _):
        pltpu.async_copy(src.at[idxv.at[i]], buf0, sem0).wait()
        pltpu.sync_copy(buf0, out_ref.at[c, pl.ds(base + i * _GC, _GC)])
        return 0

    lax.fori_loop(0, _NCH, loop, 0)


def _gather(table, gidx):
    gidx_r = gidx.reshape(_B, _S * _NS // 128, 128)
    mesh = plsc.VectorSubcoreMesh(core_axis_name="c", subcore_axis_name="s",
                                  num_cores=2, num_subcores=16)
    f = pl.kernel(
        _gather_body,
        out_type=jax.ShapeDtypeStruct((_B, _S * _NS, _D), jnp.float32),
        mesh=mesh,
        scratch_types=[
            pltpu.VMEM((_NCH, _GC), jnp.int32),
            pltpu.VMEM((_GC, _D), jnp.float32),
            pltpu.SemaphoreType.DMA,
        ],
        compiler_params=pltpu.CompilerParams(needs_layout_passes=False),
    )
    return f(table, gidx_r)


_TS = 128


def _mlp_body(g_ref, nx_ref, w1_ref, b1_ref, w2_ref, b2_ref, w3_ref, b3_ref,
              out_ref):
    g = g_ref[0]
    nx = nx_ref[0]
    w1 = w1_ref[...]
    corr = jnp.dot(nx, w1_ref[0:3, :], preferred_element_type=jnp.float32)
    corr = jnp.reshape(
        jnp.broadcast_to(corr[:, None, :], (_TS, _NS, 64)), (_TS * _NS, 64))
    h = jnp.dot(g, w1, preferred_element_type=jnp.float32)
    h = jnp.maximum(h + b1_ref[...] - corr, 0.0)
    h = jnp.dot(h, w2_ref[...], preferred_element_type=jnp.float32)
    h = jnp.maximum(h + b2_ref[...], 0.0)
    h = jnp.dot(h, w3_ref[...], preferred_element_type=jnp.float32)
    h = jnp.maximum(h + b3_ref[...], 0.0)
    out_ref[0] = jnp.max(jnp.reshape(h, (_TS, _NS, 128)), axis=1)


def _mlp(g, new_xyz, W1p, b1, W2, b2, W3, b3):
    return pl.pallas_call(
        _mlp_body,
        grid=(_B, _S // _TS),
        in_specs=[
            pl.BlockSpec((1, _TS * _NS, _D), lambda b, t: (b, t, 0)),
            pl.BlockSpec((1, _TS, 3), lambda b, t: (b, t, 0)),
            pl.BlockSpec((_D, 64), lambda b, t: (0, 0)),
            pl.BlockSpec((1, 64), lambda b, t: (0, 0)),
            pl.BlockSpec((64, 64), lambda b, t: (0, 0)),
            pl.BlockSpec((1, 64), lambda b, t: (0, 0)),
            pl.BlockSpec((64, 128), lambda b, t: (0, 0)),
            pl.BlockSpec((1, 128), lambda b, t: (0, 0)),
        ],
        out_specs=pl.BlockSpec((1, _TS, 128), lambda b, t: (b, t, 0)),
        out_shape=jax.ShapeDtypeStruct((_B, _S, 128), jnp.float32),
    )(g, new_xyz, W1p, b1.reshape(1, 64), W2, b2.reshape(1, 64),
      W3, b3.reshape(1, 128))


def kernel(xyz, features, curvature, W1, b1, W2, b2, W3, b3):
    idx, nxyz_t, new_curv = _fps(xyz, curvature)
    new_xyz = nxyz_t.transpose(0, 2, 1)
    gidx = _ball_query(xyz, new_xyz)
    table = jnp.concatenate(
        [xyz, features.transpose(0, 2, 1),
         jnp.zeros((_B, _N, _D - 3 - _C), jnp.float32)], axis=-1)
    g = _gather(table, gidx)
    W1p = jnp.zeros((_D, 64), jnp.float32).at[0:3 + _C].set(W1)
    nf = _mlp(g, new_xyz, W1p, b1, W2, b2, W3, b3)
    return new_xyz, nf.transpose(0, 2, 1), new_curv


# speedup vs baseline: 13.2257x; 1.0701x over previous
import functools

import numpy as np

import jax
import jax.numpy as jnp
from jax import lax
from jax.experimental import pallas as pl
from jax.experimental.pallas import tpu as pltpu
from jax.experimental.pallas import tpu_sc as plsc

_B, _N, _C = 2, 8192, 32
_S, _RADIUS, _NS = 1024, 0.2, 32
_R2 = float(np.float32(0.2 ** 2))
_ROWS, _LANES = _N // 128, 128


def _fps_body(xyz_ref, curv_ref, idx_ref, nxyz_ref, ncurv_ref, pw_ref):
    planes = []
    for b in range(_B):
        px = xyz_ref[b, 0, :, :]
        py = xyz_ref[b, 1, :, :]
        pz = xyz_ref[b, 2, :, :]
        nrm = jnp.sqrt(px * px + py * py + pz * pz)
        pw_ref[b] = jnp.where(nrm > 0.7, 0.0, curv_ref[b])
        planes.append((px, py, pz, pw_ref[b]))

    rows = lax.broadcasted_iota(jnp.int32, (_ROWS, _LANES), 0)
    lanes = lax.broadcasted_iota(jnp.int32, (_ROWS, _LANES), 1)
    gidx = rows * _LANES + lanes
    lane1 = lax.broadcasted_iota(jnp.int32, (1, _LANES), 1)

    def pick(plane_ref, pre, r, l):
        rowv = plane_ref[pre + (pl.ds(r, 1), slice(None))]
        return jnp.sum(jnp.where(lane1 == l, rowv, 0.0))

    def body(i, carry):
        dists, gs = carry
        new_dists, new_gs = [], []
        for b in range(_B):
            dist, g = dists[b], gs[b]
            px, py, pz, pw = planes[b]
            r = g // _LANES
            l = g % _LANES
            cx = pick(xyz_ref, (b, 0), r, l)
            cy = pick(xyz_ref, (b, 1), r, l)
            cz = pick(xyz_ref, (b, 2), r, l)
            cw = pick(pw_ref, (b,), r, l)
            idx_ref[b, 0, i] = g
            nxyz_ref[b, 0, i] = cx
            nxyz_ref[b, 1, i] = cy
            nxyz_ref[b, 2, i] = cz
            ncurv_ref[b, 0, i] = cw
            dx = px - cx
            dy = py - cy
            dz = pz - cz
            dw = pw - cw
            d = ((dx * dx + dy * dy) + dz * dz) + dw * dw
            dist = jnp.minimum(dist, d)
            m = jnp.max(dist)
            g2 = jnp.min(jnp.where(dist == m, gidx, _N))
            new_dists.append(dist)
            new_gs.append(g2)
        return tuple(new_dists), tuple(new_gs)

    dist0 = jnp.full((_ROWS, _LANES), 1e10, dtype=jnp.float32)
    lax.fori_loop(0, _S, body,
                  ((dist0,) * _B, (jnp.int32(0),) * _B))


def _fps(xyz, curv):
    xyz_pl = xyz.transpose(0, 2, 1).reshape(_B, 3, _ROWS, _LANES)
    curv_pl = curv.reshape(_B, _ROWS, _LANES)
    out = pl.pallas_call(
        _fps_body,
        in_specs=[
            pl.BlockSpec((_B, 3, _ROWS, _LANES), lambda: (0, 0, 0, 0)),
            pl.BlockSpec((_B, _ROWS, _LANES), lambda: (0, 0, 0)),
        ],
        out_specs=[
            pl.BlockSpec((_B, 1, _S), lambda: (0, 0, 0), memory_space=pltpu.SMEM),
            pl.BlockSpec((_B, 3, _S), lambda: (0, 0, 0), memory_space=pltpu.SMEM),
            pl.BlockSpec((_B, 1, _S), lambda: (0, 0, 0), memory_space=pltpu.SMEM),
        ],
        out_shape=[
            jax.ShapeDtypeStruct((_B, 1, _S), jnp.int32),
            jax.ShapeDtypeStruct((_B, 3, _S), jnp.float32),
            jax.ShapeDtypeStruct((_B, 1, _S), jnp.float32),
        ],
        scratch_shapes=[pltpu.VMEM((_B, _ROWS, _LANES), jnp.float32)],
    )(xyz_pl, curv_pl)
    return out[0].reshape(_B, _S), out[1], out[2].reshape(_B, _S)


_QPT = _S // 16
_SEG = 1024
_NSEG = _N // _SEG
_TSB = 128


def _d2_body(nx_ref, xt_ref, out_ref):
    a = nx_ref[0]
    bt = xt_ref[0]
    qx = lax.dot_general(a, bt, (((1,), (0,)), ((), ())),
                         preferred_element_type=jnp.float32)
    q2 = (a[:, 0:1] * a[:, 0:1] + a[:, 1:2] * a[:, 1:2]) + a[:, 2:3] * a[:, 2:3]
    x2 = (bt[0:1, :] * bt[0:1, :] + bt[1:2, :] * bt[1:2, :]) + bt[2:3, :] * bt[2:3, :]
    out_ref[0] = (q2 + x2) - 2.0 * qx


def _d2(new_xyz, xyz_t):
    return pl.pallas_call(
        _d2_body,
        grid=(_B, _S // _TSB),
        in_specs=[
            pl.BlockSpec((1, _TSB, 3), lambda b, t: (b, t, 0)),
            pl.BlockSpec((1, 3, _N), lambda b, t: (b, 0, 0)),
        ],
        out_specs=pl.BlockSpec((1, _TSB, _N), lambda b, t: (b, t, 0)),
        out_shape=jax.ShapeDtypeStruct((_B, _S, _N), jnp.float32),
    )(new_xyz, xyz_t)


def _ballq_body(d2_ref, out_ref, dball, dbuf, ob):
    c = lax.axis_index("c")
    s = lax.axis_index("s")
    iota = lax.iota(jnp.int32, 16)
    big = jnp.full((16,), _N, dtype=jnp.int32)
    pltpu.sync_copy(d2_ref.at[c, pl.ds(s * _QPT, _QPT), pl.ds(0, _SEG)], dball)

    def q_body(j, _):
        jv = jnp.full((16,), j, dtype=jnp.int32)
        qid = s * _QPT + j
        ob[j, pl.ds(0, 16)] = big
        ob[j, pl.ds(16, 16)] = big

        def scan_row(row_ref, row_pre, seg_base, cnt0):
            def scan(ch, cntv):
                d2c = row_ref[row_pre + (pl.ds(ch * 16, 16),)]
                m = d2c < _R2
                rank = plsc.cumsum(jnp.where(m, 1, 0))
                pos = cntv + rank - 1
                sel = m & (pos < _NS)
                lanev = (seg_base + ch * 16) + iota
                plsc.store_scatter(ob, [jv, pos], lanev, mask=sel)
                return cntv + plsc.all_reduce_population_count(m)

            cntv = lax.fori_loop(0, _SEG // 16, scan,
                                 jnp.full((16,), cnt0, dtype=jnp.int32))
            return lax.reduce_max(cntv, axes=(0,))

        cnt = scan_row(dball, (j,), jnp.int32(0), jnp.int32(0))

        def seg_cond(carry):
            seg, cnt2 = carry
            return (seg < _NSEG) & (cnt2 < _NS)

        def seg_body(carry):
            seg, cnt2 = carry
            pltpu.sync_copy(
                d2_ref.at[c, pl.ds(qid, 1), pl.ds(seg * _SEG, _SEG)], dbuf)
            cnt2 = scan_row(dbuf, (0,), seg * _SEG, cnt2)
            return seg + 1, cnt2

        _, cnt = lax.while_loop(seg_cond, seg_body, (jnp.int32(1), cnt))
        o0 = ob[j, pl.ds(0, 16)]
        o1 = ob[j, pl.ds(16, 16)]
        padv = lax.reduce_min(o0, axes=(0,))
        ob[j, pl.ds(0, 16)] = jnp.where(iota < cnt, o0, padv)
        ob[j, pl.ds(16, 16)] = jnp.where(iota + 16 < cnt, o1, padv)
        return 0

    lax.fori_loop(0, _QPT, q_body, 0)
    pltpu.sync_copy(ob, out_ref.at[c, pl.ds(s * _QPT, _QPT)])


def _ball_query(xyz, new_xyz):
    d2 = _d2(new_xyz, xyz.transpose(0, 2, 1))
    mesh = plsc.VectorSubcoreMesh(core_axis_name="c", subcore_axis_name="s",
                                  num_cores=2, num_subcores=16)
    f = pl.kernel(
        _ballq_body,
        out_type=jax.ShapeDtypeStruct((_B, _S, _NS), jnp.int32),
        mesh=mesh,
        scratch_types=[
            pltpu.VMEM((_QPT, _SEG), jnp.float32),
            pltpu.VMEM((1, _SEG), jnp.float32),
            pltpu.VMEM((_QPT, _NS), jnp.int32),
        ],
        compiler_params=pltpu.CompilerParams(needs_layout_passes=False),
    )
    return f(d2)


_D = 128
_RPT = _S * _NS // 16
_GC = 128
_NCH = _RPT // _GC


def _gather_body(tab_ref, gidx_ref, out_ref, idxv, buf0, sem0):
    c = lax.axis_index("c")
    s = lax.axis_index("s")
    pltpu.sync_copy(gidx_ref.at[c, pl.ds(s * _NCH, _NCH)], idxv)
    base = s * _RPT
    src = tab_ref.at[c]

    def loop(i, _):
        pltpu.async_copy(src.at[idxv.at[i]], buf0, sem0).wait()
        pltpu.sync_copy(buf0, out_ref.at[c, pl.ds(base + i * _GC, _GC)])
        return 0

    lax.fori_loop(0, _NCH, loop, 0)


def _gather(table, gidx):
    gidx_r = gidx.reshape(_B, _S * _NS // 128, 128)
    mesh = plsc.VectorSubcoreMesh(core_axis_name="c", subcore_axis_name="s",
                                  num_cores=2, num_subcores=16)
    f = pl.kernel(
        _gather_body,
        out_type=jax.ShapeDtypeStruct((_B, _S * _NS, _D), jnp.float32),
        mesh=mesh,
        scratch_types=[
            pltpu.VMEM((_NCH, _GC), jnp.int32),
            pltpu.VMEM((_GC, _D), jnp.float32),
            pltpu.SemaphoreType.DMA,
        ],
        compiler_params=pltpu.CompilerParams(needs_layout_passes=False),
    )
    return f(table, gidx_r)


_TS = 128


def _mlp_body(g_ref, nx_ref, w1_ref, b1_ref, w2_ref, b2_ref, w3_ref, b3_ref,
              out_ref):
    g = g_ref[0]
    nx = nx_ref[0]
    w1 = w1_ref[...]
    corr = jnp.dot(nx, w1_ref[0:3, :], preferred_element_type=jnp.float32)
    corr = jnp.reshape(
        jnp.broadcast_to(corr[:, None, :], (_TS, _NS, 64)), (_TS * _NS, 64))
    h = jnp.dot(g, w1, preferred_element_type=jnp.float32)
    h = jnp.maximum(h + b1_ref[...] - corr, 0.0)
    h = jnp.dot(h, w2_ref[...], preferred_element_type=jnp.float32)
    h = jnp.maximum(h + b2_ref[...], 0.0)
    h = jnp.dot(h, w3_ref[...], preferred_element_type=jnp.float32)
    h = jnp.maximum(h + b3_ref[...], 0.0)
    out_ref[0] = jnp.max(jnp.reshape(h, (_TS, _NS, 128)), axis=1)


def _mlp(g, new_xyz, W1p, b1, W2, b2, W3, b3):
    return pl.pallas_call(
        _mlp_body,
        grid=(_B, _S // _TS),
        in_specs=[
            pl.BlockSpec((1, _TS * _NS, _D), lambda b, t: (b, t, 0)),
            pl.BlockSpec((1, _TS, 3), lambda b, t: (b, t, 0)),
            pl.BlockSpec((_D, 64), lambda b, t: (0, 0)),
            pl.BlockSpec((1, 64), lambda b, t: (0, 0)),
            pl.BlockSpec((64, 64), lambda b, t: (0, 0)),
            pl.BlockSpec((1, 64), lambda b, t: (0, 0)),
            pl.BlockSpec((64, 128), lambda b, t: (0, 0)),
            pl.BlockSpec((1, 128), lambda b, t: (0, 0)),
        ],
        out_specs=pl.BlockSpec((1, _TS, 128), lambda b, t: (b, t, 0)),
        out_shape=jax.ShapeDtypeStruct((_B, _S, 128), jnp.float32),
    )(g, new_xyz, W1p, b1.reshape(1, 64), W2, b2.reshape(1, 64),
      W3, b3.reshape(1, 128))


def kernel(xyz, features, curvature, W1, b1, W2, b2, W3, b3):
    idx, nxyz_t, new_curv = _fps(xyz, curvature)
    new_xyz = nxyz_t.transpose(0, 2, 1)
    gidx = _ball_query(xyz, new_xyz)
    table = jnp.concatenate(
        [xyz, features.transpose(0, 2, 1),
         jnp.zeros((_B, _N, _D - 3 - _C), jnp.float32)], axis=-1)
    g = _gather(table, gidx)
    W1p = jnp.zeros((_D, 64), jnp.float32).at[0:3 + _C].set(W1)
    nf = _mlp(g, new_xyz, W1p, b1, W2, b2, W3, b3)
    return new_xyz, nf.transpose(0, 2, 1), new_curv
